# baseline (speedup 1.0000x reference)
import functools

import jax
import jax.numpy as jnp
from jax import lax
from jax.experimental import pallas as pl
from jax.experimental.pallas import tpu as pltpu
from jax.experimental.pallas import tpu_sc as plsc

_CHUNK = 128


@functools.cache
def _build_gather(V, D, B, base):
    info = plsc.get_sparse_core_info()
    NC, NS = info.num_cores, info.num_subcores
    NW = NC * NS
    b_per_w = B // NW
    assert B % (NW * _CHUNK) == 0
    n_chunks = b_per_w // _CHUNK
    n_streams = 2 * n_chunks
    NBUF = 3

    mesh = plsc.VectorSubcoreMesh(core_axis_name="c", subcore_axis_name="s")

    @functools.partial(
        pl.kernel,
        mesh=mesh,
        out_type=(
            jax.ShapeDtypeStruct((B, D), jnp.float32),
            jax.ShapeDtypeStruct((B, D), jnp.float32),
        ),
        scratch_types=[
            pltpu.VMEM((n_streams * _CHUNK,), jnp.int32),
            pltpu.VMEM((NBUF, _CHUNK, D), jnp.float32),
            pltpu.SemaphoreType.DMA,
            pltpu.SemaphoreType.DMA,
            pltpu.SemaphoreType.DMA,
        ],
    )
    def gather(cidx, didx, ctab, dtab, outc, outd, idx_v, rows_v, s0, s1, s2):
        wid = lax.axis_index("s") * NC + lax.axis_index("c")
        sems = (s0, s1, s2)
        pltpu.sync_copy(
            cidx.at[pl.ds(base + wid * b_per_w, b_per_w)],
            idx_v.at[pl.ds(0, b_per_w)],
        )
        pltpu.sync_copy(
            didx.at[pl.ds(base + wid * b_per_w, b_per_w)],
            idx_v.at[pl.ds(b_per_w, b_per_w)],
        )

        def out_slice(g):
            t, j = g // n_chunks, g % n_chunks
            out = (outc, outd)[t]
            base = wid * b_per_w + j * _CHUNK
            return out.at[pl.ds(base, _CHUNK)]

        cps = [None] * NBUF
        for g in range(n_streams):
            tab = (ctab, dtab)[g // n_chunks]
            slot = g % NBUF
            cps[slot] = pltpu.async_copy(
                tab.at[idx_v.at[pl.ds(g * _CHUNK, _CHUNK)]], rows_v.at[slot], sems[slot]
            )
            if g >= 1:
                prev = (g - 1) % NBUF
                cps[prev].wait()
                pltpu.sync_copy(rows_v.at[prev], out_slice(g - 1))
        last = (n_streams - 1) % NBUF
        cps[last].wait()
        pltpu.sync_copy(rows_v.at[last], out_slice(n_streams - 1))

    return gather


def _mlp(zc, zd, W1, b1, W2, b2):
    B, D = zc.shape
    BLK = 2048

    def body(zc_ref, zd_ref, w1_ref, b1_ref, w2_ref, b2_ref, o_ref):
        h = jnp.dot(
            zc_ref[...], w1_ref[0:D, :], preferred_element_type=jnp.float32
        )
        h = h + jnp.dot(
            zd_ref[...], w1_ref[D : 2 * D, :], preferred_element_type=jnp.float32
        )
        h = jnp.maximum(h + b1_ref[...], 0.0)
        o = jnp.dot(h, w2_ref[...], preferred_element_type=jnp.float32) + b2_ref[0]
        o_ref[...] = o.reshape(BLK // 128, 128)

    out = pl.pallas_call(
        body,
        grid=(B // BLK,),
        in_specs=[
            pl.BlockSpec((BLK, D), lambda i: (i, 0)),
            pl.BlockSpec((BLK, D), lambda i: (i, 0)),
            pl.BlockSpec((2 * D, D), lambda i: (0, 0)),
            pl.BlockSpec((D,), lambda i: (0,)),
            pl.BlockSpec((D, 1), lambda i: (0, 0)),
            pl.BlockSpec((1,), lambda i: (0,)),
        ],
        out_specs=pl.BlockSpec((BLK // 128, 128), lambda i: (i, 0)),
        out_shape=jax.ShapeDtypeStruct((B // 128, 128), jnp.float32),
        compiler_params=pltpu.CompilerParams(
            dimension_semantics=("parallel",)
        ),
    )(zc, zd, W1, b1, W2, b2)
    return out.reshape(B)


_NSPLIT = 1


def kernel(compound_idx, disease_idx, compound_table, disease_table, W1, b1, W2, b2):
    V, D = compound_table.shape
    B = compound_idx.shape[0]
    bs = B // _NSPLIT
    zs = []
    for k in range(_NSPLIT):
        gather = _build_gather(V, D, bs, k * bs)
        zs.append(gather(compound_idx, disease_idx, compound_table, disease_table))
    outs = [_mlp(zc, zd, W1, b1, W2, b2) for zc, zd in zs]
    return jnp.concatenate(outs)


# speedup vs baseline: 1.0197x; 1.0197x over previous
import functools

import jax
import jax.numpy as jnp
from jax import lax
from jax.experimental import pallas as pl
from jax.experimental.pallas import tpu as pltpu
from jax.experimental.pallas import tpu_sc as plsc

_CHUNK = 128


@functools.cache
def _build_gather(V, D, B, base):
    info = plsc.get_sparse_core_info()
    NC, NS = info.num_cores, info.num_subcores
    NW = NC * NS
    b_per_w = B // NW
    assert B % (NW * _CHUNK) == 0
    n_chunks = b_per_w // _CHUNK
    n_streams = 2 * n_chunks
    NBUF = 3

    mesh = plsc.VectorSubcoreMesh(core_axis_name="c", subcore_axis_name="s")

    @functools.partial(
        pl.kernel,
        mesh=mesh,
        out_type=(
            jax.ShapeDtypeStruct((B, D), jnp.float32),
            jax.ShapeDtypeStruct((B, D), jnp.float32),
        ),
        scratch_types=[
            pltpu.VMEM((n_streams * _CHUNK,), jnp.int32),
            pltpu.VMEM((NBUF, _CHUNK, D), jnp.float32),
            pltpu.SemaphoreType.DMA,
            pltpu.SemaphoreType.DMA,
            pltpu.SemaphoreType.DMA,
            pltpu.SemaphoreType.DMA,
            pltpu.SemaphoreType.DMA,
            pltpu.SemaphoreType.DMA,
        ],
    )
    def gather(
        cidx, didx, ctab, dtab, outc, outd, idx_v, rows_v, s0, s1, s2, t0, t1, t2
    ):
        wid = lax.axis_index("s") * NC + lax.axis_index("c")
        sems = (s0, s1, s2)
        osems = (t0, t1, t2)
        pltpu.sync_copy(
            cidx.at[pl.ds(base + wid * b_per_w, b_per_w)],
            idx_v.at[pl.ds(0, b_per_w)],
        )
        pltpu.sync_copy(
            didx.at[pl.ds(base + wid * b_per_w, b_per_w)],
            idx_v.at[pl.ds(b_per_w, b_per_w)],
        )

        def out_slice(g):
            t, j = g // n_chunks, g % n_chunks
            out = (outc, outd)[t]
            base = wid * b_per_w + j * _CHUNK
            return out.at[pl.ds(base, _CHUNK)]

        cps = [None] * NBUF
        ocps = [None] * NBUF
        for g in range(n_streams):
            tab = (ctab, dtab)[g // n_chunks]
            slot = g % NBUF
            if g >= NBUF:
                ocps[slot].wait()
            cps[slot] = pltpu.async_copy(
                tab.at[idx_v.at[pl.ds(g * _CHUNK, _CHUNK)]], rows_v.at[slot], sems[slot]
            )
            if g >= 1:
                prev = (g - 1) % NBUF
                cps[prev].wait()
                ocps[prev] = pltpu.async_copy(
                    rows_v.at[prev], out_slice(g - 1), osems[prev]
                )
        last = (n_streams - 1) % NBUF
        cps[last].wait()
        ocps[last] = pltpu.async_copy(rows_v.at[last], out_slice(n_streams - 1), osems[last])
        for slot in range(NBUF):
            ocps[slot].wait()

    return gather


def _mlp(zc, zd, W1, b1, W2, b2):
    B, D = zc.shape
    BLK = 2048

    def body(zc_ref, zd_ref, w1_ref, b1_ref, w2_ref, b2_ref, o_ref):
        h = jnp.dot(
            zc_ref[...], w1_ref[0:D, :], preferred_element_type=jnp.float32
        )
        h = h + jnp.dot(
            zd_ref[...], w1_ref[D : 2 * D, :], preferred_element_type=jnp.float32
        )
        h = jnp.maximum(h + b1_ref[...], 0.0)
        o = jnp.dot(h, w2_ref[...], preferred_element_type=jnp.float32) + b2_ref[0]
        o_ref[...] = o.reshape(BLK // 128, 128)

    out = pl.pallas_call(
        body,
        grid=(B // BLK,),
        in_specs=[
            pl.BlockSpec((BLK, D), lambda i: (i, 0)),
            pl.BlockSpec((BLK, D), lambda i: (i, 0)),
            pl.BlockSpec((2 * D, D), lambda i: (0, 0)),
            pl.BlockSpec((D,), lambda i: (0,)),
            pl.BlockSpec((D, 1), lambda i: (0, 0)),
            pl.BlockSpec((1,), lambda i: (0,)),
        ],
        out_specs=pl.BlockSpec((BLK // 128, 128), lambda i: (i, 0)),
        out_shape=jax.ShapeDtypeStruct((B // 128, 128), jnp.float32),
        compiler_params=pltpu.CompilerParams(
            dimension_semantics=("parallel",)
        ),
    )(zc, zd, W1, b1, W2, b2)
    return out.reshape(B)


_NSPLIT = 1


def kernel(compound_idx, disease_idx, compound_table, disease_table, W1, b1, W2, b2):
    V, D = compound_table.shape
    B = compound_idx.shape[0]
    bs = B // _NSPLIT
    zs = []
    for k in range(_NSPLIT):
        gather = _build_gather(V, D, bs, k * bs)
        zs.append(gather(compound_idx, disease_idx, compound_table, disease_table))
    outs = [_mlp(zc, zd, W1, b1, W2, b2) for zc, zd in zs]
    return jnp.concatenate(outs)


# speedup vs baseline: 1.0716x; 1.0509x over previous
import functools

import jax
import jax.numpy as jnp
from jax import lax
from jax.experimental import pallas as pl
from jax.experimental.pallas import tpu as pltpu
from jax.experimental.pallas import tpu_sc as plsc

_CHUNK = 128


@functools.cache
def _build_gather(V, D, B, base):
    info = plsc.get_sparse_core_info()
    NC, NS = info.num_cores, info.num_subcores
    NW = NC * NS
    b_per_w = B // NW
    assert B % (NW * _CHUNK) == 0
    n_chunks = b_per_w // _CHUNK
    n_streams = 2 * n_chunks
    NBUF = 3

    mesh = plsc.VectorSubcoreMesh(core_axis_name="c", subcore_axis_name="s")

    @functools.partial(
        pl.kernel,
        mesh=mesh,
        out_type=(
            jax.ShapeDtypeStruct((B, D), jnp.float32),
            jax.ShapeDtypeStruct((B, D), jnp.float32),
        ),
        scratch_types=[
            pltpu.VMEM((n_streams * _CHUNK,), jnp.int32),
            pltpu.VMEM((NBUF, _CHUNK, D), jnp.float32),
            pltpu.SemaphoreType.DMA,
            pltpu.SemaphoreType.DMA,
            pltpu.SemaphoreType.DMA,
            pltpu.SemaphoreType.DMA,
            pltpu.SemaphoreType.DMA,
            pltpu.SemaphoreType.DMA,
        ],
    )
    def gather(
        cidx, didx, ctab, dtab, outc, outd, idx_v, rows_v, s0, s1, s2, t0, t1, t2
    ):
        wid = lax.axis_index("s") * NC + lax.axis_index("c")
        sems = (s0, s1, s2)
        osems = (t0, t1, t2)
        pltpu.sync_copy(
            cidx.at[pl.ds(base + wid * b_per_w, b_per_w)],
            idx_v.at[pl.ds(0, b_per_w)],
        )
        pltpu.sync_copy(
            didx.at[pl.ds(base + wid * b_per_w, b_per_w)],
            idx_v.at[pl.ds(b_per_w, b_per_w)],
        )

        def out_slice(g):
            t, j = g // n_chunks, g % n_chunks
            out = (outc, outd)[t]
            base = wid * b_per_w + j * _CHUNK
            return out.at[pl.ds(base, _CHUNK)]

        cps = [None] * NBUF
        ocps = [None] * NBUF
        for g in range(n_streams):
            tab = (ctab, dtab)[g // n_chunks]
            slot = g % NBUF
            if g >= NBUF:
                ocps[slot].wait()
            cps[slot] = pltpu.async_copy(
                tab.at[idx_v.at[pl.ds(g * _CHUNK, _CHUNK)]], rows_v.at[slot], sems[slot]
            )
            if g >= 1:
                prev = (g - 1) % NBUF
                cps[prev].wait()
                ocps[prev] = pltpu.async_copy(
                    rows_v.at[prev], out_slice(g - 1), osems[prev]
                )
        last = (n_streams - 1) % NBUF
        cps[last].wait()
        ocps[last] = pltpu.async_copy(rows_v.at[last], out_slice(n_streams - 1), osems[last])
        for slot in range(NBUF):
            ocps[slot].wait()

    return gather


def _mlp(zc, zd, W1, b1, W2, b2):
    B, D = zc.shape
    BLK = 4096

    def body(zc_ref, zd_ref, w1_ref, b1_ref, w2_ref, b2_ref, o_ref):
        h = jnp.dot(
            zc_ref[...], w1_ref[0:D, :], preferred_element_type=jnp.float32
        )
        h = h + jnp.dot(
            zd_ref[...], w1_ref[D : 2 * D, :], preferred_element_type=jnp.float32
        )
        h = jnp.maximum(h + b1_ref[...], 0.0)
        o = jnp.dot(h, w2_ref[...], preferred_element_type=jnp.float32) + b2_ref[0]
        o_ref[...] = o.reshape(BLK // 128, 128)

    out = pl.pallas_call(
        body,
        grid=(B // BLK,),
        in_specs=[
            pl.BlockSpec((BLK, D), lambda i: (i, 0)),
            pl.BlockSpec((BLK, D), lambda i: (i, 0)),
            pl.BlockSpec((2 * D, D), lambda i: (0, 0)),
            pl.BlockSpec((D,), lambda i: (0,)),
            pl.BlockSpec((D, 1), lambda i: (0, 0)),
            pl.BlockSpec((1,), lambda i: (0,)),
        ],
        out_specs=pl.BlockSpec((BLK // 128, 128), lambda i: (i, 0)),
        out_shape=jax.ShapeDtypeStruct((B // 128, 128), jnp.float32),
        compiler_params=pltpu.CompilerParams(
            dimension_semantics=("parallel",)
        ),
    )(zc, zd, W1, b1, W2, b2)
    return out.reshape(B)


_NSPLIT = 1


def kernel(compound_idx, disease_idx, compound_table, disease_table, W1, b1, W2, b2):
    V, D = compound_table.shape
    B = compound_idx.shape[0]
    bs = B // _NSPLIT
    zs = []
    for k in range(_NSPLIT):
        gather = _build_gather(V, D, bs, k * bs)
        zs.append(gather(compound_idx, disease_idx, compound_table, disease_table))
    outs = [_mlp(zc, zd, W1, b1, W2, b2) for zc, zd in zs]
    return jnp.concatenate(outs)


# speedup vs baseline: 1.0942x; 1.0211x over previous
import functools

import jax
import jax.numpy as jnp
from jax import lax
from jax.experimental import pallas as pl
from jax.experimental.pallas import tpu as pltpu
from jax.experimental.pallas import tpu_sc as plsc

_CHUNK = 128


@functools.cache
def _build_gather(V, D, B, base):
    info = plsc.get_sparse_core_info()
    NC, NS = info.num_cores, info.num_subcores
    NW = NC * NS
    b_per_w = B // NW
    assert B % (NW * _CHUNK) == 0
    n_chunks = b_per_w // _CHUNK
    n_streams = 2 * n_chunks
    NBUF = 3

    mesh = plsc.VectorSubcoreMesh(core_axis_name="c", subcore_axis_name="s")

    @functools.partial(
        pl.kernel,
        mesh=mesh,
        out_type=(
            jax.ShapeDtypeStruct((B, D), jnp.float32),
            jax.ShapeDtypeStruct((B, D), jnp.float32),
        ),
        scratch_types=[
            pltpu.VMEM((n_streams * _CHUNK,), jnp.int32),
            pltpu.VMEM((NBUF, _CHUNK, D), jnp.float32),
            pltpu.SemaphoreType.DMA,
            pltpu.SemaphoreType.DMA,
            pltpu.SemaphoreType.DMA,
            pltpu.SemaphoreType.DMA,
            pltpu.SemaphoreType.DMA,
            pltpu.SemaphoreType.DMA,
        ],
    )
    def gather(
        cidx, didx, ctab, dtab, outc, outd, idx_v, rows_v, s0, s1, s2, t0, t1, t2
    ):
        wid = lax.axis_index("s") * NC + lax.axis_index("c")
        sems = (s0, s1, s2)
        osems = (t0, t1, t2)
        pltpu.sync_copy(
            cidx.at[pl.ds(base + wid * b_per_w, b_per_w)],
            idx_v.at[pl.ds(0, b_per_w)],
        )
        pltpu.sync_copy(
            didx.at[pl.ds(base + wid * b_per_w, b_per_w)],
            idx_v.at[pl.ds(b_per_w, b_per_w)],
        )

        def out_slice(g):
            t, j = g // n_chunks, g % n_chunks
            out = (outc, outd)[t]
            base = wid * b_per_w + j * _CHUNK
            return out.at[pl.ds(base, _CHUNK)]

        cps = [None] * NBUF
        ocps = [None] * NBUF
        for g in range(n_streams):
            tab = (ctab, dtab)[g // n_chunks]
            slot = g % NBUF
            if g >= NBUF:
                ocps[slot].wait()
            cps[slot] = pltpu.async_copy(
                tab.at[idx_v.at[pl.ds(g * _CHUNK, _CHUNK)]], rows_v.at[slot], sems[slot]
            )
            if g >= 1:
                prev = (g - 1) % NBUF
                cps[prev].wait()
                ocps[prev] = pltpu.async_copy(
                    rows_v.at[prev], out_slice(g - 1), osems[prev]
                )
        last = (n_streams - 1) % NBUF
        cps[last].wait()
        ocps[last] = pltpu.async_copy(rows_v.at[last], out_slice(n_streams - 1), osems[last])
        for slot in range(NBUF):
            ocps[slot].wait()

    return gather


def _mlp(zc, zd, W1, b1, W2, b2):
    B, D = zc.shape
    BLK = 8192

    def body(zc_ref, zd_ref, w1_ref, b1_ref, w2_ref, b2_ref, o_ref):
        h = jnp.dot(
            zc_ref[...], w1_ref[0:D, :], preferred_element_type=jnp.float32
        )
        h = h + jnp.dot(
            zd_ref[...], w1_ref[D : 2 * D, :], preferred_element_type=jnp.float32
        )
        h = jnp.maximum(h + b1_ref[...], 0.0)
        o = jnp.dot(h, w2_ref[...], preferred_element_type=jnp.float32) + b2_ref[0]
        o_ref[...] = o.reshape(BLK // 128, 128)

    out = pl.pallas_call(
        body,
        grid=(B // BLK,),
        in_specs=[
            pl.BlockSpec((BLK, D), lambda i: (i, 0)),
            pl.BlockSpec((BLK, D), lambda i: (i, 0)),
            pl.BlockSpec((2 * D, D), lambda i: (0, 0)),
            pl.BlockSpec((D,), lambda i: (0,)),
            pl.BlockSpec((D, 1), lambda i: (0, 0)),
            pl.BlockSpec((1,), lambda i: (0,)),
        ],
        out_specs=pl.BlockSpec((BLK // 128, 128), lambda i: (i, 0)),
        out_shape=jax.ShapeDtypeStruct((B // 128, 128), jnp.float32),
        compiler_params=pltpu.CompilerParams(
            dimension_semantics=("parallel",)
        ),
    )(zc, zd, W1, b1, W2, b2)
    return out.reshape(B)


_NSPLIT = 1


def kernel(compound_idx, disease_idx, compound_table, disease_table, W1, b1, W2, b2):
    V, D = compound_table.shape
    B = compound_idx.shape[0]
    bs = B // _NSPLIT
    zs = []
    for k in range(_NSPLIT):
        gather = _build_gather(V, D, bs, k * bs)
        zs.append(gather(compound_idx, disease_idx, compound_table, disease_table))
    outs = [_mlp(zc, zd, W1, b1, W2, b2) for zc, zd in zs]
    return jnp.concatenate(outs)
